# baseline (speedup 1.0000x reference)
import functools

import jax
import jax.numpy as jnp
from jax import lax
from jax.experimental import pallas as pl
from jax.experimental.pallas import tpu as pltpu
from jax.experimental.pallas import tpu_sc as plsc

_N = 10000
_E = 320000
_D = 128
_H = 64
_G = 64
_EPS = 1e-5

_NC = 2
_NS = 16
_SBE = 500
_NSB = _E // _SBE
_SBT = _NSB // _NS
_NP = 10240
_RPT = _NP // _NS


def _sc_scatter_body(h0_hbm, h1_hbm, src_hbm, dst_hbm, out_hbm,
                     si0, si1, si2, si3, di0, di1, di2, di3,
                     rw0, rw1, acc, gs0, gs1, is0, is1, is2, is3):
    c = lax.axis_index("c")
    s = lax.axis_index("s")
    r0 = s * _RPT
    sidx = (si0, si1, si2, si3)
    didx = (di0, di1, di2, di3)
    rows = (rw0, rw1)
    gsem = (gs0, gs1)
    isem = (is0, is1, is2, is3)

    with jax.named_scope("sc_zero"):
        zv = jnp.zeros((16,), jnp.float32)

        def zbody(i, carry):
            for jj in range(_H // 16):
                rw0[i, pl.ds(jj * 16, 16)] = zv
            return carry

        lax.fori_loop(0, _RPT // 2, zbody, 0)
        pltpu.sync_copy(rw0.at[pl.ds(0, _RPT // 2), :],
                        acc.at[pl.ds(r0, _RPT // 2), :])
        pltpu.sync_copy(rw0.at[pl.ds(0, _RPT // 2), :],
                        acc.at[pl.ds(r0 + _RPT // 2, _RPT // 2), :])
        plsc.subcore_barrier()

    def sb_of(j):
        return s + j * _NS

    for core, h_hbm in ((0, h0_hbm), (1, h1_hbm)):
        @pl.when(c == core)
        def _(h_hbm=h_hbm):
            def idx_load(j, p, sync=False):
                m = p % 4
                if sync:
                    pltpu.sync_copy(src_hbm.at[sb_of(j)], sidx[m])
                    pltpu.sync_copy(dst_hbm.at[sb_of(j)], didx[m])
                else:
                    pltpu.async_copy(src_hbm.at[sb_of(j)], sidx[m], isem[m])
                    pltpu.async_copy(dst_hbm.at[sb_of(j)], didx[m], isem[m])

            def idx_wait(j, p):
                m = p % 4
                pltpu.make_async_copy(src_hbm.at[sb_of(j)], sidx[m],
                                      isem[m]).wait()
                pltpu.make_async_copy(dst_hbm.at[sb_of(j)], didx[m],
                                      isem[m]).wait()

            def gather(j, p):
                pltpu.async_copy(h_hbm.at[sidx[p % 4]], rows[p % 2],
                                 gsem[p % 2])

            def gather_wait(j, p):
                pltpu.make_async_copy(h_hbm.at[sidx[p % 4]], rows[p % 2],
                                      gsem[p % 2]).wait()

            def scat(j, p):
                with jax.named_scope("sc_scat"):
                    pltpu.sync_copy(rows[p % 2], acc.at[didx[p % 4]],
                                    add=True)

            def step(j, p, load=True):
                gather_wait(j, p)
                scat(j, p)
                if load:
                    idx_load(j + 2, p + 2)
                idx_wait(j + 1, p + 1)
                gather(j + 1, p + 1)

            idx_load(0, 0, sync=True)
            idx_load(1, 1)
            gather(0, 0)

            def body(k, carry):
                j0 = 4 * k
                step(j0, 0)
                step(j0 + 1, 1)
                step(j0 + 2, 2)
                step(j0 + 3, 3)
                return carry

            lax.fori_loop(0, _SBT // 4 - 1, body, 0)
            for j in range(_SBT - 4, _SBT - 1):
                step(j, j % 4, load=(j + 2 < _SBT))
            gather_wait(_SBT - 1, (_SBT - 1) % 4)
            scat(_SBT - 1, (_SBT - 1) % 4)

    plsc.subcore_barrier()
    with jax.named_scope("sc_out"):
        pltpu.sync_copy(acc.at[pl.ds(r0, _RPT), :],
                        out_hbm.at[c, pl.ds(r0, _RPT), :])


@functools.cache
def _get_sc_scatter():
    return pl.kernel(
        _sc_scatter_body,
        out_type=jax.ShapeDtypeStruct((_NC, _NP, _H), jnp.float32),
        mesh=plsc.VectorSubcoreMesh(core_axis_name="c", subcore_axis_name="s"),
        compiler_params=pltpu.CompilerParams(use_tc_tiling_on_sc=False),
        scratch_types=(
            [pltpu.VMEM((_SBE,), jnp.int32)] * 8
            + [pltpu.VMEM((_SBE, _H), jnp.float32)] * 2
            + [pltpu.VMEM_SHARED((_NP, _H), jnp.float32)]
            + [pltpu.SemaphoreType.DMA] * 6
        ),
    )


def _tc_layer_body(final, ha_ref, hb_ref, aggr_ref, batch_ref,
                   vw_ref, vb_ref, aw_ref, ab_ref, rw_ref, rb_ref,
                   g_ref, b_ref, lw_ref, lb_ref, *out_refs):
    h = jnp.concatenate([ha_ref[...], hb_ref[...]], axis=1)
    aggr = jnp.concatenate([aggr_ref[0, :_N], aggr_ref[1, :_N]], axis=1)
    onehot = (batch_ref[...] ==
              lax.broadcasted_iota(jnp.int32, (_N, _G), 1)).astype(jnp.float32)
    pooled = lax.dot_general(onehot, h, (((0,), (0,)), ((), ())),
                             preferred_element_type=jnp.float32)
    pr = jnp.dot(pooled, rw_ref[...], preferred_element_type=jnp.float32)
    comb = (jnp.dot(h, vw_ref[...], preferred_element_type=jnp.float32)
            + jnp.dot(aggr, aw_ref[...], preferred_element_type=jnp.float32)
            + jnp.dot(onehot, pr, preferred_element_type=jnp.float32)
            + vb_ref[...] + ab_ref[...] + rb_ref[...])
    hr = jnp.maximum(comb, 0.0)
    mean = jnp.mean(hr, axis=0, keepdims=True)
    var = jnp.mean((hr - mean) * (hr - mean), axis=0, keepdims=True)
    hn = (hr - mean) * lax.rsqrt(var + _EPS) * g_ref[...] + b_ref[...]
    if final:
        out_refs[0][...] = (jnp.dot(hn, lw_ref[...],
                                    preferred_element_type=jnp.float32)
                            + lb_ref[...])
    else:
        out_refs[0][...] = hn[:, :_H]
        out_refs[1][...] = hn[:, _H:]


def _tc_layer(final, ha, hb, aggr, batch_col,
              vw, vb, aw, ab, rw, rb, g, b, lw, lb):
    if final:
        out_shape = jax.ShapeDtypeStruct((_N, lw.shape[1]), jnp.float32)
    else:
        out_shape = (jax.ShapeDtypeStruct((_N, _H), jnp.float32),
                     jax.ShapeDtypeStruct((_N, _H), jnp.float32))
    return pl.pallas_call(
        functools.partial(_tc_layer_body, final),
        out_shape=out_shape,
    )(ha, hb, aggr, batch_col, vw, vb.reshape(1, -1), aw, ab.reshape(1, -1),
      rw, rb.reshape(1, -1), g.reshape(1, -1), b.reshape(1, -1),
      lw, lb.reshape(1, -1))


def kernel(x, edge_index, batch,
           V0w, V0b, A0w, A0b, R0w, R0b, bn0_g, bn0_b,
           V1w, V1b, A1w, A1b, R1w, R1b, bn1_g, bn1_b,
           lin_w, lin_b):
    src = edge_index[0].reshape(_NSB, _SBE)
    dst = edge_index[1].reshape(_NSB, _SBE)
    batch_col = batch.reshape(_N, 1)
    xa = x[:, :_H]
    xb = x[:, _H:]

    sc_scatter = _get_sc_scatter()
    aggr0 = sc_scatter(xa, xb, src, dst)
    h1a, h1b = _tc_layer(False, xa, xb, aggr0, batch_col,
                         V0w, V0b, A0w, A0b, R0w, R0b, bn0_g, bn0_b,
                         lin_w, lin_b)
    aggr1 = sc_scatter(h1a, h1b, src, dst)
    out = _tc_layer(True, h1a, h1b, aggr1, batch_col,
                    V1w, V1b, A1w, A1b, R1w, R1b, bn1_g, bn1_b, lin_w, lin_b)
    return out


# speedup vs baseline: 1.2801x; 1.2801x over previous
import functools

import jax
import jax.numpy as jnp
from jax import lax
from jax.experimental import pallas as pl
from jax.experimental.pallas import tpu as pltpu
from jax.experimental.pallas import tpu_sc as plsc

_N = 10000
_E = 320000
_D = 128
_H = 64
_G = 64
_EPS = 1e-5

_NC = 2
_NS = 16
_SBE = 500
_NSB = _E // _SBE
_SBT = _NSB // _NS
_NP = 10240
_RPT = _NP // _NS


def _sc_scatter_body(h0_hbm, h1_hbm, src_hbm, dst_hbm, out_hbm,
                     si0, si1, si2, si3, di0, di1, di2, di3,
                     rw0, rw1, acc, gs0, gs1, is0, is1, is2, is3):
    c = lax.axis_index("c")
    s = lax.axis_index("s")
    r0 = s * _RPT
    sidx = (si0, si1, si2, si3)
    didx = (di0, di1, di2, di3)
    rows = (rw0, rw1)
    gsem = (gs0, gs1)
    isem = (is0, is1, is2, is3)

    with jax.named_scope("sc_zero"):
        zv = jnp.zeros((16,), jnp.float32)

        def zbody(i, carry):
            for jj in range(_H // 16):
                rw0[i, pl.ds(jj * 16, 16)] = zv
            return carry

        lax.fori_loop(0, _RPT // 2, zbody, 0)
        pltpu.sync_copy(rw0.at[pl.ds(0, _RPT // 2), :],
                        acc.at[pl.ds(r0, _RPT // 2), :])
        pltpu.sync_copy(rw0.at[pl.ds(0, _RPT // 2), :],
                        acc.at[pl.ds(r0 + _RPT // 2, _RPT // 2), :])
        plsc.subcore_barrier()

    def sb_of(j):
        return s + j * _NS

    for core, h_hbm in ((0, h0_hbm), (1, h1_hbm)):
        @pl.when(c == core)
        def _(h_hbm=h_hbm):
            def idx_load(j, p, sync=False):
                m = p % 4
                if sync:
                    pltpu.sync_copy(src_hbm.at[sb_of(j)], sidx[m])
                    pltpu.sync_copy(dst_hbm.at[sb_of(j)], didx[m])
                else:
                    pltpu.async_copy(src_hbm.at[sb_of(j)], sidx[m], isem[m])
                    pltpu.async_copy(dst_hbm.at[sb_of(j)], didx[m], isem[m])

            def idx_wait(j, p):
                m = p % 4
                pltpu.make_async_copy(src_hbm.at[sb_of(j)], sidx[m],
                                      isem[m]).wait()
                pltpu.make_async_copy(dst_hbm.at[sb_of(j)], didx[m],
                                      isem[m]).wait()

            def gather(j, p):
                pltpu.async_copy(h_hbm.at[sidx[p % 4]], rows[p % 2],
                                 gsem[p % 2])

            def gather_wait(j, p):
                pltpu.make_async_copy(h_hbm.at[sidx[p % 4]], rows[p % 2],
                                      gsem[p % 2]).wait()

            def scat(j, p):
                with jax.named_scope("sc_scat"):
                    pltpu.sync_copy(rows[p % 2], acc.at[didx[p % 4]],
                                    add=True)

            def step(j, p, load=True, gath=True):
                gather_wait(j, p)
                scat(j, p)
                if load:
                    idx_load(j + 3, p + 3)
                if gath:
                    idx_wait(j + 2, p + 2)
                    gather(j + 2, p + 2)

            idx_load(0, 0, sync=True)
            idx_load(1, 1)
            idx_load(2, 2)
            gather(0, 0)
            idx_wait(1, 1)
            gather(1, 1)

            def body(k, carry):
                j0 = 4 * k
                step(j0, 0)
                step(j0 + 1, 1)
                step(j0 + 2, 2)
                step(j0 + 3, 3)
                return carry

            lax.fori_loop(0, _SBT // 4 - 1, body, 0)
            for j in range(_SBT - 4, _SBT):
                step(j, j % 4, load=(j + 3 < _SBT), gath=(j + 2 < _SBT))

    plsc.subcore_barrier()
    with jax.named_scope("sc_out"):
        pltpu.sync_copy(acc.at[pl.ds(r0, _RPT), :],
                        out_hbm.at[c, pl.ds(r0, _RPT), :])


@functools.cache
def _get_sc_scatter():
    return pl.kernel(
        _sc_scatter_body,
        out_type=jax.ShapeDtypeStruct((_NC, _NP, _H), jnp.float32),
        mesh=plsc.VectorSubcoreMesh(core_axis_name="c", subcore_axis_name="s"),
        compiler_params=pltpu.CompilerParams(use_tc_tiling_on_sc=False),
        scratch_types=(
            [pltpu.VMEM((_SBE,), jnp.int32)] * 8
            + [pltpu.VMEM((_SBE, _H), jnp.float32)] * 2
            + [pltpu.VMEM_SHARED((_NP, _H), jnp.float32)]
            + [pltpu.SemaphoreType.DMA] * 6
        ),
    )


def _tc_layer_body(final, ha_ref, hb_ref, aggr_ref, batch_ref,
                   vw_ref, vb_ref, aw_ref, ab_ref, rw_ref, rb_ref,
                   g_ref, b_ref, lw_ref, lb_ref, *out_refs):
    h = jnp.concatenate([ha_ref[...], hb_ref[...]], axis=1)
    aggr = jnp.concatenate([aggr_ref[0, :_N], aggr_ref[1, :_N]], axis=1)
    onehot = (batch_ref[...] ==
              lax.broadcasted_iota(jnp.int32, (_N, _G), 1)).astype(jnp.float32)
    pooled = lax.dot_general(onehot, h, (((0,), (0,)), ((), ())),
                             preferred_element_type=jnp.float32)
    pr = jnp.dot(pooled, rw_ref[...], preferred_element_type=jnp.float32)
    comb = (jnp.dot(h, vw_ref[...], preferred_element_type=jnp.float32)
            + jnp.dot(aggr, aw_ref[...], preferred_element_type=jnp.float32)
            + jnp.dot(onehot, pr, preferred_element_type=jnp.float32)
            + vb_ref[...] + ab_ref[...] + rb_ref[...])
    hr = jnp.maximum(comb, 0.0)
    mean = jnp.mean(hr, axis=0, keepdims=True)
    var = jnp.mean((hr - mean) * (hr - mean), axis=0, keepdims=True)
    hn = (hr - mean) * lax.rsqrt(var + _EPS) * g_ref[...] + b_ref[...]
    if final:
        out_refs[0][...] = (jnp.dot(hn, lw_ref[...],
                                    preferred_element_type=jnp.float32)
                            + lb_ref[...])
    else:
        out_refs[0][...] = hn[:, :_H]
        out_refs[1][...] = hn[:, _H:]


def _tc_layer(final, ha, hb, aggr, batch_col,
              vw, vb, aw, ab, rw, rb, g, b, lw, lb):
    if final:
        out_shape = jax.ShapeDtypeStruct((_N, lw.shape[1]), jnp.float32)
    else:
        out_shape = (jax.ShapeDtypeStruct((_N, _H), jnp.float32),
                     jax.ShapeDtypeStruct((_N, _H), jnp.float32))
    return pl.pallas_call(
        functools.partial(_tc_layer_body, final),
        out_shape=out_shape,
    )(ha, hb, aggr, batch_col, vw, vb.reshape(1, -1), aw, ab.reshape(1, -1),
      rw, rb.reshape(1, -1), g.reshape(1, -1), b.reshape(1, -1),
      lw, lb.reshape(1, -1))


def kernel(x, edge_index, batch,
           V0w, V0b, A0w, A0b, R0w, R0b, bn0_g, bn0_b,
           V1w, V1b, A1w, A1b, R1w, R1b, bn1_g, bn1_b,
           lin_w, lin_b):
    src = edge_index[0].reshape(_NSB, _SBE)
    dst = edge_index[1].reshape(_NSB, _SBE)
    batch_col = batch.reshape(_N, 1)
    xa = x[:, :_H]
    xb = x[:, _H:]

    sc_scatter = _get_sc_scatter()
    aggr0 = sc_scatter(xa, xb, src, dst)
    h1a, h1b = _tc_layer(False, xa, xb, aggr0, batch_col,
                         V0w, V0b, A0w, A0b, R0w, R0b, bn0_g, bn0_b,
                         lin_w, lin_b)
    aggr1 = sc_scatter(h1a, h1b, src, dst)
    out = _tc_layer(True, h1a, h1b, aggr1, batch_col,
                    V1w, V1b, A1w, A1b, R1w, R1b, bn1_g, bn1_b, lin_w, lin_b)
    return out
